# Initial kernel scaffold; baseline (speedup 1.0000x reference)
import jax
import jax.numpy as jnp
from jax.experimental import pallas as pl


def kernel(x, conv1, ln1_g, ln1_b, s0b0_w1, s0b0_g1, s0b0_b1, s0b0_w2, s0b0_g2, s0b0_b2, s0b0_wd, s0b0_gd, s0b0_bd, s0b1_w1, s0b1_g1, s0b1_b1, s0b1_w2, s0b1_g2, s0b1_b2, s1b0_w1, s1b0_g1, s1b0_b1, s1b0_w2, s1b0_g2, s1b0_b2, s1b0_wd, s1b0_gd, s1b0_bd, s1b1_w1, s1b1_g1, s1b1_b1, s1b1_w2, s1b1_g2, s1b1_b2, s2b0_w1, s2b0_g1, s2b0_b1, s2b0_w2, s2b0_g2, s2b0_b2, s2b0_wd, s2b0_gd, s2b0_bd, s2b1_w1, s2b1_g1, s2b1_b1, s2b1_w2, s2b1_g2, s2b1_b2, s3b0_w1, s3b0_g1, s3b0_b1, s3b0_w2, s3b0_g2, s3b0_b2, s3b0_wd, s3b0_gd, s3b0_bd, s3b1_w1, s3b1_g1, s3b1_b1, s3b1_w2, s3b1_g2, s3b1_b2):
    raise NotImplementedError("write your pallas kernel here")


# speedup vs baseline: 15.1349x; 15.1349x over previous
import jax
import jax.numpy as jnp
from jax.experimental import pallas as pl
from jax.experimental.pallas import tpu as pltpu

EPS = 1e-5
LANES = 128
VMEM_LIMIT = 32 * 1024 * 1024


def _round8(n):
    return -(-n // 8) * 8


def _cmask(c_true):
    lane = jax.lax.broadcasted_iota(jnp.int32, (1, LANES), 1)
    return (lane < c_true).astype(jnp.float32)


def _ln(acc, c_true, gamma, beta, cmask):
    inv_c = 1.0 / float(c_true)
    acc = acc * cmask
    mu = jnp.sum(acc, axis=-1, keepdims=True) * inv_c
    xc = (acc - mu) * cmask
    var = jnp.sum(xc * xc, axis=-1, keepdims=True) * inv_c
    return xc * jax.lax.rsqrt(var + EPS) * gamma + beta


def _conv3x3_from_buf(buf_ref, src_bf16, taps_ref, rows, spatial, pad,
                      hrow, wcol):
    buf_ref[pad:pad + rows, :] = src_bf16
    acc = jnp.zeros((rows, LANES), jnp.float32)
    for t in range(9):
        di, dj = t // 3, t % 3
        off = (di - 1) * spatial + (dj - 1)
        tap = buf_ref[pad + off:pad + off + rows, :]
        hh = hrow + (di - 1)
        ww = wcol + (dj - 1)
        valid = ((hh >= 0) & (hh < spatial) &
                 (ww >= 0) & (ww < spatial)).astype(jnp.float32)
        acc = acc + valid * jnp.dot(tap, taps_ref[t],
                                    preferred_element_type=jnp.float32)
    return acc


_S_IN = 35
_S_OUT = 32
_M_STEM = _S_OUT * _S_IN


def _stem_body(xs_ref, w_ref, g_ref, b_ref, o_ref):
    cm = _cmask(64)
    acc = jnp.zeros((_M_STEM, LANES), jnp.float32)
    for t in range(16):
        a, b = t // 4, t % 4
        off = a * _S_IN + b
        acc = acc + jnp.dot(xs_ref[0, off:off + _M_STEM, :], w_ref[t],
                            preferred_element_type=jnp.float32)
    y = jnp.maximum(_ln(acc, 64, g_ref[...], b_ref[...], cm), 0.0)
    o_ref[0] = y[:, :64]


def _stem(xs, w4, g, b, n):
    return pl.pallas_call(
        _stem_body,
        out_shape=jax.ShapeDtypeStruct((n, _M_STEM, 64), jnp.float32),
        grid=(n,),
        in_specs=[
            pl.BlockSpec((1,) + xs.shape[1:], lambda i: (i, 0, 0)),
            pl.BlockSpec(w4.shape, lambda i: (0, 0, 0)),
            pl.BlockSpec((1, LANES), lambda i: (0, 0)),
            pl.BlockSpec((1, LANES), lambda i: (0, 0)),
        ],
        out_specs=pl.BlockSpec((1, _M_STEM, 64), lambda i: (i, 0, 0)),
        compiler_params=pltpu.CompilerParams(
            dimension_semantics=("parallel",),
            vmem_limit_bytes=VMEM_LIMIT),
    )(xs, w4, g, b)


def _make_stage_body(c_out, spatial, rows, pad, conv1_taps):
    def _body(*refs):
        if conv1_taps:
            (x_ref, w1t_ref, g1_ref, b1_ref, w2t_ref, g2_ref, b2_ref,
             wd_ref, gd_ref, bd_ref,
             v1t_ref, h1_ref, c1_ref, v2t_ref, h2_ref, c2_ref,
             o_ref, buf_ref) = refs
        else:
            (a1_ref, ad_ref, w1_ref, g1_ref, b1_ref, w2t_ref, g2_ref,
             b2_ref, wd_ref, gd_ref, bd_ref,
             v1t_ref, h1_ref, c1_ref, v2t_ref, h2_ref, c2_ref,
             o_ref, buf_ref) = refs

        cm = _cmask(c_out)
        buf_ref[0:pad, :] = jnp.zeros((pad, LANES), jnp.bfloat16)
        buf_ref[pad + rows:pad + rows + pad, :] = (
            jnp.zeros((pad, LANES), jnp.bfloat16))

        ridx = jax.lax.broadcasted_iota(jnp.int32, (rows, 1), 0)
        wcol = ridx % spatial
        hrow = (ridx // spatial) % spatial

        if conv1_taps:
            acc1 = _conv3x3_from_buf(buf_ref, x_ref[...], w1t_ref, rows,
                                     spatial, pad, hrow, wcol)
        else:
            acc1 = jnp.dot(a1_ref[...], w1_ref[...],
                           preferred_element_type=jnp.float32)
        out1 = jnp.maximum(_ln(acc1, c_out, g1_ref[...], b1_ref[...], cm),
                           0.0)

        acc2 = _conv3x3_from_buf(buf_ref, out1.astype(jnp.bfloat16),
                                 w2t_ref, rows, spatial, pad, hrow, wcol)
        y2 = _ln(acc2, c_out, g2_ref[...], b2_ref[...], cm)
        ds_in = x_ref[...] if conv1_taps else ad_ref[...]
        accd = jnp.dot(ds_in, wd_ref[...],
                       preferred_element_type=jnp.float32)
        yd = _ln(accd, c_out, gd_ref[...], bd_ref[...], cm)
        r0 = jnp.maximum(y2 + yd, 0.0)

        acc3 = _conv3x3_from_buf(buf_ref, r0.astype(jnp.bfloat16),
                                 v1t_ref, rows, spatial, pad, hrow, wcol)
        out3 = jnp.maximum(_ln(acc3, c_out, h1_ref[...], c1_ref[...], cm),
                           0.0)
        acc4 = _conv3x3_from_buf(buf_ref, out3.astype(jnp.bfloat16),
                                 v2t_ref, rows, spatial, pad, hrow, wcol)
        y4 = _ln(acc4, c_out, h2_ref[...], c2_ref[...], cm)
        o_ref[...] = jnp.maximum(y4 + r0, 0.0)[:, :c_out]

    return _body


def _run_stage(args, m, rows, c_out, spatial, conv1_taps):
    pad = _round8(spatial + 1)
    grid = m // rows
    n_part = 1 if conv1_taps else 2
    in_specs = []
    for ai, a in enumerate(args):
        if ai < n_part:
            in_specs.append(
                pl.BlockSpec((rows, a.shape[1]), lambda i: (i, 0)))
        elif a.ndim == 3:
            in_specs.append(pl.BlockSpec(a.shape, lambda i: (0, 0, 0)))
        else:
            in_specs.append(pl.BlockSpec(a.shape, lambda i: (0, 0)))
    return pl.pallas_call(
        _make_stage_body(c_out, spatial, rows, pad, conv1_taps),
        out_shape=jax.ShapeDtypeStruct((m, c_out), jnp.float32),
        grid=(grid,),
        in_specs=in_specs,
        out_specs=pl.BlockSpec((rows, c_out), lambda i: (i, 0)),
        scratch_shapes=[pltpu.VMEM((rows + 2 * pad, LANES), jnp.bfloat16)],
        compiler_params=pltpu.CompilerParams(
            dimension_semantics=("parallel",),
            vmem_limit_bytes=VMEM_LIMIT),
    )(*args)


def _pack_w(w_oihw):
    cout, cin, kh, kw = w_oihw.shape
    wt = jnp.transpose(w_oihw, (2, 3, 1, 0)).astype(jnp.float32)
    wt = wt.reshape(kh * kw * cin, cout)
    return jnp.pad(wt, ((0, 0), (0, LANES - cout))).astype(jnp.bfloat16)


def _pack_w_taps(w_oihw):
    cout, cin, _, _ = w_oihw.shape
    wt = jnp.transpose(w_oihw, (2, 3, 1, 0)).astype(jnp.float32)
    wt = wt.reshape(9, cin, cout)
    wt = jnp.pad(wt, ((0, 0), (0, LANES - cin), (0, LANES - cout)))
    return wt.astype(jnp.bfloat16)


def _pack_wd(w_oihw, k_pad):
    cout, cin, _, _ = w_oihw.shape
    wt = jnp.transpose(w_oihw[:, :, 0, 0], (1, 0)).astype(jnp.float32)
    return jnp.pad(wt, ((0, k_pad - cin), (0, LANES - cout))).astype(
        jnp.bfloat16)


def _pad_vec(v):
    c = v.shape[0]
    return jnp.pad(v.astype(jnp.float32), (0, LANES - c)).reshape(1, LANES)


def _im2col_s2(x_nhwc):
    n, h, w, c = x_nhwc.shape
    xp = jnp.pad(x_nhwc, ((0, 0), (1, 1), (1, 1), (0, 0)))
    oh, ow = h // 2, w // 2
    taps = [xp[:, i:i + 2 * oh - 1:2, j:j + 2 * ow - 1:2, :]
            for i in range(3) for j in range(3)]
    cols = jnp.concatenate(taps, axis=-1).reshape(n * oh * ow, 9 * c)
    return cols.astype(jnp.bfloat16)


def _maxpool3x3s2(x_nhwc):
    n, h, w, c = x_nhwc.shape
    oh, ow = (h - 1) // 2 + 1, (w - 1) // 2 + 1
    xp = jnp.pad(x_nhwc, ((0, 0), (1, 1), (1, 1), (0, 0)))
    out = None
    for i in range(3):
        for j in range(3):
            t = xp[:, i:i + 2 * (oh - 1) + 1:2, j:j + 2 * (ow - 1) + 1:2, :]
            out = t if out is None else jnp.maximum(out, t)
    return out


def _stem_s2d(x_nchw):
    x = jnp.transpose(x_nchw, (0, 2, 3, 1))
    n = x.shape[0]
    xp = jnp.pad(x, ((0, 0), (3, 3), (3, 3), (0, 0)))
    xs = xp.reshape(n, _S_IN, 2, _S_IN, 2, 96)
    xs = jnp.transpose(xs, (0, 1, 3, 2, 4, 5))
    xs = xs.reshape(n, _S_IN * _S_IN, 384)
    xs = jnp.pad(xs, ((0, 0), (0, 1232 - _S_IN * _S_IN), (0, 0)))
    return xs.astype(jnp.bfloat16)


def _stem_w4(conv1):
    wp = jnp.pad(conv1.astype(jnp.float32),
                 ((0, 0), (0, 0), (0, 1), (0, 1)))
    wp = wp.reshape(64, 96, 4, 2, 4, 2)
    wp = jnp.transpose(wp, (2, 4, 3, 5, 1, 0))
    wp = wp.reshape(16, 384, 64)
    return jnp.pad(wp, ((0, 0), (0, 0), (0, LANES - 64))).astype(jnp.bfloat16)


_STAGE_ROWS = (512, 256, 128, 32)


def kernel(x, conv1, ln1_g, ln1_b, s0b0_w1, s0b0_g1, s0b0_b1, s0b0_w2, s0b0_g2, s0b0_b2, s0b0_wd, s0b0_gd, s0b0_bd, s0b1_w1, s0b1_g1, s0b1_b1, s0b1_w2, s0b1_g2, s0b1_b2, s1b0_w1, s1b0_g1, s1b0_b1, s1b0_w2, s1b0_g2, s1b0_b2, s1b0_wd, s1b0_gd, s1b0_bd, s1b1_w1, s1b1_g1, s1b1_b1, s1b1_w2, s1b1_g2, s1b1_b2, s2b0_w1, s2b0_g1, s2b0_b1, s2b0_w2, s2b0_g2, s2b0_b2, s2b0_wd, s2b0_gd, s2b0_bd, s2b1_w1, s2b1_g1, s2b1_b1, s2b1_w2, s2b1_g2, s2b1_b2, s3b0_w1, s3b0_g1, s3b0_b1, s3b0_w2, s3b0_g2, s3b0_b2, s3b0_wd, s3b0_gd, s3b0_bd, s3b1_w1, s3b1_g1, s3b1_b1, s3b1_w2, s3b1_g2, s3b1_b2):
    n = x.shape[0]

    y = _stem(_stem_s2d(x), _stem_w4(conv1), _pad_vec(ln1_g),
              _pad_vec(ln1_b), n)
    y = y.reshape(n, _S_OUT, _S_IN, 64)[:, :, :_S_OUT, :]
    xs0 = _maxpool3x3s2(y)

    stages = [
        (s0b0_w1, s0b0_g1, s0b0_b1, s0b0_w2, s0b0_g2, s0b0_b2,
         s0b0_wd, s0b0_gd, s0b0_bd,
         s0b1_w1, s0b1_g1, s0b1_b1, s0b1_w2, s0b1_g2, s0b1_b2),
        (s1b0_w1, s1b0_g1, s1b0_b1, s1b0_w2, s1b0_g2, s1b0_b2,
         s1b0_wd, s1b0_gd, s1b0_bd,
         s1b1_w1, s1b1_g1, s1b1_b1, s1b1_w2, s1b1_g2, s1b1_b2),
        (s2b0_w1, s2b0_g1, s2b0_b1, s2b0_w2, s2b0_g2, s2b0_b2,
         s2b0_wd, s2b0_gd, s2b0_bd,
         s2b1_w1, s2b1_g1, s2b1_b1, s2b1_w2, s2b1_g2, s2b1_b2),
        (s3b0_w1, s3b0_g1, s3b0_b1, s3b0_w2, s3b0_g2, s3b0_b2,
         s3b0_wd, s3b0_gd, s3b0_bd,
         s3b1_w1, s3b1_g1, s3b1_b1, s3b1_w2, s3b1_g2, s3b1_b2),
    ]

    outs = []
    cur = xs0
    for si, p in enumerate(stages):
        (w1, g1, b1, w2, g2, b2, wd, gd, bd,
         v1, h1, c1, v2, h2, c2) = p
        c_out = w1.shape[0]
        cin = w1.shape[1]
        spatial = cur.shape[1] if si == 0 else cur.shape[1] // 2
        m = n * spatial * spatial
        common = [_pad_vec(g1), _pad_vec(b1), _pack_w_taps(w2),
                  _pad_vec(g2), _pad_vec(b2)]
        ds = [_pad_vec(gd), _pad_vec(bd)]
        blk1 = [_pack_w_taps(v1), _pad_vec(h1), _pad_vec(c1),
                _pack_w_taps(v2), _pad_vec(h2), _pad_vec(c2)]
        if si == 0:
            xin = jnp.pad(cur.reshape(m, cin),
                          ((0, 0), (0, LANES - cin))).astype(jnp.bfloat16)
            args = ([xin, _pack_w_taps(w1)] + common
                    + [_pack_wd(wd, LANES), ] + ds + blk1)
        else:
            a1 = _im2col_s2(cur)
            ad = cur[:, ::2, ::2, :].reshape(m, cin).astype(jnp.bfloat16)
            args = ([a1, ad, _pack_w(w1)] + common
                    + [_pack_wd(wd, cin), ] + ds + blk1)
        o = _run_stage(args, m, min(_STAGE_ROWS[si], m), c_out, spatial,
                       si == 0)
        cur = o.reshape(n, spatial, spatial, c_out)
        outs.append(jnp.transpose(cur, (0, 3, 1, 2)))
    return tuple(outs)
